# baseline (speedup 1.0000x reference)
import functools

import jax
import jax.numpy as jnp
from jax import lax
from jax.experimental import pallas as pl
from jax.experimental.pallas import tpu as pltpu
from jax.experimental.pallas import tpu_sc as plsc

_VOCAB = 119
_D = 512
_BATCH, _SEQ = 4096, 50
_SEQP = 56

_NC, _NS = 2, 16
_NW = _NC * _NS
_B_PER_W = _BATCH // _NW
_RING = 4


def _proj_body(t1, w1, b1, t2, w2, b2, t3, w3, b3, o1, o2, o3):
    o1[...] = jnp.dot(t1[...], w1[...], preferred_element_type=jnp.float32) + b1[...]
    o2[...] = jnp.dot(t2[...], w2[...], preferred_element_type=jnp.float32) + b2[...]
    o3[...] = jnp.dot(t3[...], w3[...], preferred_element_type=jnp.float32) + b3[...]


def _project_tables(t1, w1, b1, t2, w2, b2, t3, w3, b3):
    out = [jax.ShapeDtypeStruct((_VOCAB, _D), jnp.float32)] * 3
    return pl.pallas_call(_proj_body, out_shape=out)(
        t1, w1, b1.reshape(1, _D), t2, w2, b2.reshape(1, _D),
        t3, w3, b3.reshape(1, _D))


_mesh = plsc.VectorSubcoreMesh(core_axis_name="c", subcore_axis_name="s")


@functools.partial(
    pl.kernel,
    mesh=_mesh,
    out_type=jax.ShapeDtypeStruct((_BATCH, _SEQ, _D), jnp.float32),
    scratch_types=[
        pltpu.VMEM((_B_PER_W, _SEQP), jnp.int32),
        pltpu.VMEM((_SEQ, _D), jnp.float32),
        pltpu.VMEM((_SEQ, _D), jnp.float32),
        pltpu.VMEM((_SEQ, _D), jnp.float32),
        pltpu.VMEM((_SEQ, _D), jnp.float32),
        pltpu.SemaphoreType.DMA,
        pltpu.SemaphoreType.DMA,
    ],
)
def _gather_one(tbl, idx_hbm, out, idx_v, r0, r1, r2, r3, gsem, wsem):
    wid = lax.axis_index("s") * _NC + lax.axis_index("c")
    bufs = (r0, r1, r2, r3)

    b0 = wid * _B_PER_W
    pltpu.sync_copy(idx_hbm.at[pl.ds(b0, _B_PER_W)], idx_v)

    def ring_body(c, carry):
        for j in range(_RING):
            b = c * _RING + j
            @pl.when(c > 0)
            def _drain():
                pltpu.make_async_copy(bufs[j], out.at[0], wsem).wait()

            idx_c = idx_v.at[b, pl.ds(0, _SEQ)]
            pltpu.async_copy(tbl.at[idx_c], bufs[j], gsem).wait()
            pltpu.async_copy(bufs[j], out.at[b0 + b], wsem)
        return carry

    lax.fori_loop(0, _B_PER_W // _RING, ring_body, 0)
    for j in range(_RING):
        pltpu.make_async_copy(bufs[j], out.at[0], wsem).wait()


def kernel(Z, table_mat2vec, table_magpie, table_oliynyk,
           W_mat2vec, b_mat2vec, W_magpie, b_magpie, W_oliynyk, b_oliynyk):
    p1, p2, p3 = _project_tables(
        table_mat2vec, W_mat2vec, b_mat2vec,
        table_magpie, W_magpie, b_magpie,
        table_oliynyk, W_oliynyk, b_oliynyk)
    zp = jnp.pad(Z, ((0, 0), (0, _SEQP - _SEQ)))
    o1 = _gather_one(p1, zp)
    o2 = _gather_one(p2, zp)
    o3 = _gather_one(p3, zp)
    return (o1, o2, o3)
